# baseline (speedup 1.0000x reference)
import functools

import jax
import jax.numpy as jnp
from jax import lax
from jax.experimental import pallas as pl
from jax.experimental.pallas import tpu as pltpu
from jax.experimental.pallas import tpu_sc as plsc

_STAT_MEDIAN = 0.0
_STAT_MAX = 1.0
_STAT_MIN = -1.0
_SCALE = 2.0 / (_STAT_MAX - _STAT_MIN)

_N_NODES = 10000
_D = 128
_E = 320000

_NC = 2
_NS = 16
_NW = _NC * _NS
_E_PER_W = _E // _NW
_CHUNK = 80
_N_CHUNKS = _E_PER_W // _CHUNK
_NBUF = 5
_IDXW = 10112


@functools.partial(
    pl.kernel,
    mesh=plsc.VectorSubcoreMesh(core_axis_name="c", subcore_axis_name="s"),
    out_type=jax.ShapeDtypeStruct((_E, _D), jnp.float32),
    scratch_types=(
        [pltpu.VMEM((_IDXW,), jnp.int32)] * 2
        + [pltpu.VMEM((_CHUNK, _D), jnp.float32)] * (2 * _NBUF)
        + [pltpu.SemaphoreType.DMA] * (2 * _NBUF)
    ),
)
def _edge_diff(table_hbm, ei_hbm, out_hbm, src_v, dst_v,
               u0, v0, u1, v1, u2, v2, u3, v3, u4, v4,
               g0, g1, g2, g3, g4, w0, w1, w2, w3, w4):
    cid = lax.axis_index("c")
    sid = lax.axis_index("s")
    wid = sid * _NC + cid
    base = wid * _E_PER_W

    start = pl.multiple_of((base // 128) * 128, 128)
    shift = base - start
    pltpu.sync_copy(ei_hbm.at[0, 0, pl.ds(start, _IDXW)], src_v)
    pltpu.sync_copy(ei_hbm.at[1, 0, pl.ds(start, _IDXW)], dst_v)

    bufs = ((u0, v0, g0, w0), (u1, v1, g1, w1), (u2, v2, g2, w2),
            (u3, v3, g3, w3), (u4, v4, g4, w4))

    def start_gather(c, k):
        ub, vb, g, _w = bufs[k]
        off = shift + c * _CHUNK
        pltpu.async_copy(table_hbm.at[src_v.at[pl.ds(off, _CHUNK)]], ub, g)
        pltpu.async_copy(table_hbm.at[dst_v.at[pl.ds(off, _CHUNK)]], vb, g)

    def wait_gather(c, k):
        ub, vb, g, _w = bufs[k]
        off = shift + c * _CHUNK
        pltpu.make_async_copy(
            table_hbm.at[src_v.at[pl.ds(off, _CHUNK)]], ub, g).wait()
        pltpu.make_async_copy(
            table_hbm.at[dst_v.at[pl.ds(off, _CHUNK)]], vb, g).wait()

    def start_write(c, k):
        ub, _v, _g, w = bufs[k]
        pltpu.async_copy(ub, out_hbm.at[pl.ds(base + c * _CHUNK, _CHUNK)], w)

    def wait_write(k):
        ub, _v, _g, w = bufs[k]
        pltpu.make_async_copy(ub, out_hbm.at[pl.ds(base, _CHUNK)], w).wait()

    def compute(k):
        ub, vb, _g, _w = bufs[k]

        def row(i, carry):
            for j in range(_D // 16):
                sl = pl.ds(j * 16, 16)
                ub[i, sl] = (ub[i, sl] - vb[i, sl]) * _SCALE
            return carry

        lax.fori_loop(0, _CHUNK, row, 0)

    start_gather(0, 0)
    start_gather(1, 1)
    start_gather(2, 2)

    def body(i, carry):
        for k in range(_NBUF):
            cc = _NBUF * i + k
            kn = (k + 3) % _NBUF
            pl.when(cc >= 2)(lambda: wait_write(kn))
            pl.when(cc + 3 < _N_CHUNKS)(lambda: start_gather(cc + 3, kn))
            wait_gather(cc, k)
            compute(k)
            start_write(cc, k)
        return carry

    lax.fori_loop(0, _N_CHUNKS // _NBUF, body, 0)

    wait_write(3)
    wait_write(4)


def _norm_body(x_ref, o_ref):
    o_ref[...] = (x_ref[...] - _STAT_MEDIAN) * _SCALE


_norm = pl.pallas_call(
    _norm_body,
    out_shape=jax.ShapeDtypeStruct((_N_NODES, _D), jnp.float32),
    grid=(5,),
    in_specs=[pl.BlockSpec((_N_NODES // 5, _D), lambda i: (i, 0))],
    out_specs=pl.BlockSpec((_N_NODES // 5, _D), lambda i: (i, 0)),
)


def kernel(node_feature, edge_index):
    ei = edge_index.astype(jnp.int32).reshape(2, 1, _E)
    edge_feature = _edge_diff(node_feature, ei)
    norm = _norm(node_feature)
    return (norm, edge_feature)
